# baseline (speedup 1.0000x reference)
import functools

import jax
import jax.numpy as jnp
from jax import lax
from jax.experimental import pallas as pl
from jax.experimental.pallas import tpu as pltpu
from jax.experimental.pallas import tpu_sc as plsc

_N = 8192
_NW = 32
_RPW = _N // _NW
_RCHUNK = 4
_NCELL = 256
_HALF = _NCELL // 2


def _build_luts(boundaries, data_type):
    k = jnp.arange(_NCELL, dtype=jnp.float32)
    left = k / _HALF - 1.0
    right = (k + 1.0) / _HALF - 1.0
    b = boundaries[None, :]
    idx_left = jnp.sum((b < left[:, None]).astype(jnp.int32), axis=1)
    idx_right = jnp.sum((b < right[:, None]).astype(jnp.int32), axis=1)
    has = idx_right - idx_left
    inb = (b >= left[:, None]) & (b < right[:, None])
    th = jnp.sum(jnp.where(inb, b, 0.0), axis=1)
    thresh = jnp.where(has > 0, th, 2.0).astype(jnp.float32)
    lo = data_type[idx_left]
    hi = data_type[idx_left + has]
    return thresh, lo, hi


def kernel(x, boundaries, data_type):
    thresh, lo, hi = _build_luts(boundaries, data_type)
    mesh = plsc.VectorSubcoreMesh(core_axis_name="c", subcore_axis_name="s")

    @functools.partial(
        pl.kernel,
        mesh=mesh,
        out_type=jax.ShapeDtypeStruct((_N, _N), jnp.float32),
        compiler_params=pltpu.CompilerParams(needs_layout_passes=False),
        scratch_types=[
            pltpu.VMEM((_NCELL,), jnp.float32),
            pltpu.VMEM((_NCELL,), jnp.float32),
            pltpu.VMEM((_NCELL,), jnp.float32),
            pltpu.VMEM((_RCHUNK, _N), jnp.float32),
            pltpu.VMEM((_RCHUNK, _N), jnp.float32),
        ],
    )
    def sck(x_hbm, th_hbm, lo_hbm, hi_hbm, o_hbm, th_v, lo_v, hi_v, xin, xout):
        wid = lax.axis_index("s") * 2 + lax.axis_index("c")
        pltpu.sync_copy(th_hbm, th_v)
        pltpu.sync_copy(lo_hbm, lo_v)
        pltpu.sync_copy(hi_hbm, hi_v)
        base0 = wid * _RPW

        @pl.loop(0, _RPW // _RCHUNK)
        def _chunk(chunk_i):
            base = base0 + chunk_i * _RCHUNK
            pltpu.sync_copy(x_hbm.at[pl.ds(base, _RCHUNK)], xin)
            for r in range(_RCHUNK):
                def amax_body(c, acc):
                    for u in range(4):
                        v = xin[r, pl.ds(c * 64 + u * 16, 16)]
                        acc = jnp.maximum(acc, jnp.abs(v))
                    return acc

                acc = lax.fori_loop(0, _N // 64, amax_body,
                                    jnp.zeros((16,), jnp.float32))
                s = jnp.maximum(jnp.max(acc), 1e-6)
                sv = lax.broadcast(s, (16,))
                inv = 1.0 / sv

                @pl.loop(0, _N, step=64)
                def _q(c):
                    for u in range(4):
                        v = xin[r, pl.ds(c + u * 16, 16)]
                        xn = v * inv
                        cf = xn * jnp.float32(_HALF) + jnp.float32(_HALF)
                        cell = jnp.minimum(cf.astype(jnp.int32), _NCELL - 1)
                        t = plsc.load_gather(th_v, [cell])
                        lov = plsc.load_gather(lo_v, [cell])
                        hiv = plsc.load_gather(hi_v, [cell])
                        val = jnp.where(xn > t, hiv, lov)
                        xout[r, pl.ds(c + u * 16, 16)] = val * s

            pltpu.sync_copy(xout, o_hbm.at[pl.ds(base, _RCHUNK)])

    return sck(x, thresh, lo, hi)


# speedup vs baseline: 3.1843x; 3.1843x over previous
import functools

import jax
import jax.numpy as jnp
from jax import lax
from jax.experimental import pallas as pl
from jax.experimental.pallas import tpu as pltpu
from jax.experimental.pallas import tpu_sc as plsc

_N = 8192
_NW = 32
_RPW = _N // _NW
_RCHUNK = 4
_NCELL = 256
_HALF = _NCELL // 2


def _build_luts(boundaries, data_type):
    k = jnp.arange(_NCELL, dtype=jnp.float32)
    left = k / _HALF - 1.0
    right = (k + 1.0) / _HALF - 1.0
    b = boundaries[None, :]
    idx_left = jnp.sum((b < left[:, None]).astype(jnp.int32), axis=1)
    idx_right = jnp.sum((b < right[:, None]).astype(jnp.int32), axis=1)
    has = idx_right - idx_left
    inb = (b >= left[:, None]) & (b < right[:, None])
    th = jnp.sum(jnp.where(inb, b, 0.0), axis=1)
    thresh = jnp.where(has > 0, th, 2.0).astype(jnp.float32)
    lo = data_type[idx_left]
    hi = data_type[idx_left + has]
    return thresh, lo, hi


def kernel(x, boundaries, data_type):
    thresh, lo, hi = _build_luts(boundaries, data_type)
    mesh = plsc.VectorSubcoreMesh(core_axis_name="c", subcore_axis_name="s")

    @functools.partial(
        pl.kernel,
        mesh=mesh,
        out_type=jax.ShapeDtypeStruct((_N, _N), jnp.float32),
        compiler_params=pltpu.CompilerParams(needs_layout_passes=False),
        scratch_types=[
            pltpu.VMEM((_NCELL,), jnp.float32),
            pltpu.VMEM((_NCELL,), jnp.float32),
            pltpu.VMEM((_NCELL,), jnp.float32),
            pltpu.VMEM((_RCHUNK, _N), jnp.float32),
            pltpu.VMEM((_RCHUNK, _N), jnp.float32),
        ],
    )
    def sck(x_hbm, th_hbm, lo_hbm, hi_hbm, o_hbm, th_v, lo_v, hi_v, xin, xout):
        wid = lax.axis_index("s") * 2 + lax.axis_index("c")
        pltpu.sync_copy(th_hbm, th_v)
        pltpu.sync_copy(lo_hbm, lo_v)
        pltpu.sync_copy(hi_hbm, hi_v)
        base0 = wid * _RPW

        @pl.loop(0, _RPW // _RCHUNK)
        def _chunk(chunk_i):
            base = base0 + chunk_i * _RCHUNK
            pltpu.sync_copy(x_hbm.at[pl.ds(base, _RCHUNK)], xin)
            for r in range(_RCHUNK):
                zero = jnp.zeros((16,), jnp.float32)

                @plsc.parallel_loop(0, _N, 64, unroll=2,
                                    carry=(zero, zero, zero, zero))
                def accs(c, carry):
                    a0, a1, a2, a3 = carry
                    a0 = jnp.maximum(a0, jnp.abs(xin[r, pl.ds(c, 16)]))
                    a1 = jnp.maximum(a1, jnp.abs(xin[r, pl.ds(c + 16, 16)]))
                    a2 = jnp.maximum(a2, jnp.abs(xin[r, pl.ds(c + 32, 16)]))
                    a3 = jnp.maximum(a3, jnp.abs(xin[r, pl.ds(c + 48, 16)]))
                    return (a0, a1, a2, a3)

                acc = jnp.maximum(jnp.maximum(accs[0], accs[1]),
                                  jnp.maximum(accs[2], accs[3]))
                s = jnp.maximum(jnp.max(acc), 1e-6)
                sv = lax.broadcast(s, (16,))
                inv = 1.0 / sv

                @plsc.parallel_loop(0, _N, 16, unroll=8)
                def _q(c):
                    v = xin[r, pl.ds(c, 16)]
                    xn = v * inv
                    cf = xn * jnp.float32(_HALF) + jnp.float32(_HALF)
                    cell = jnp.minimum(cf.astype(jnp.int32), _NCELL - 1)
                    t = plsc.load_gather(th_v, [cell])
                    lov = plsc.load_gather(lo_v, [cell])
                    hiv = plsc.load_gather(hi_v, [cell])
                    val = jnp.where(xn > t, hiv, lov)
                    xout[r, pl.ds(c, 16)] = val * s

            pltpu.sync_copy(xout, o_hbm.at[pl.ds(base, _RCHUNK)])

    return sck(x, thresh, lo, hi)


# speedup vs baseline: 4.4670x; 1.4028x over previous
import functools

import jax
import jax.numpy as jnp
from jax import lax
from jax.experimental import pallas as pl
from jax.experimental.pallas import tpu as pltpu
from jax.experimental.pallas import tpu_sc as plsc

_N = 8192
_RCHUNK = 2
_NCELL = 256
_HALF = _NCELL // 2


def _build_luts(boundaries, data_type):
    k = jnp.arange(_NCELL, dtype=jnp.float32)
    left = k / _HALF - 1.0
    right = (k + 1.0) / _HALF - 1.0
    b = boundaries[None, :]
    idx_left = jnp.sum((b < left[:, None]).astype(jnp.int32), axis=1)
    idx_right = jnp.sum((b < right[:, None]).astype(jnp.int32), axis=1)
    has = idx_right - idx_left
    inb = (b >= left[:, None]) & (b < right[:, None])
    th = jnp.sum(jnp.where(inb, b, 0.0), axis=1)
    thresh = jnp.where(has > 0, th, 2.0).astype(jnp.float32)
    lo = data_type[idx_left]
    hi = data_type[idx_left + has]
    return thresh, lo, hi


def kernel(x, boundaries, data_type):
    thresh, lo, hi = _build_luts(boundaries, data_type)
    mesh = plsc.VectorSubcoreMesh(core_axis_name="c", subcore_axis_name="s")

    @functools.partial(
        pl.kernel,
        mesh=mesh,
        out_type=jax.ShapeDtypeStruct((_N, _N), jnp.float32),
        compiler_params=pltpu.CompilerParams(needs_layout_passes=False),
        scratch_types=[
            pltpu.VMEM((_NCELL,), jnp.float32),
            pltpu.VMEM((_NCELL,), jnp.float32),
            pltpu.VMEM((_NCELL,), jnp.float32),
        ],
    )
    def sck(x_hbm, th_hbm, lo_hbm, hi_hbm, o_hbm, th_v, lo_v, hi_v):
        pltpu.sync_copy(th_hbm, th_v)
        pltpu.sync_copy(lo_hbm, lo_v)
        pltpu.sync_copy(hi_hbm, hi_v)

        def body(in_v, out_v):
            for r in range(_RCHUNK):
                zero = jnp.zeros((16,), jnp.float32)

                @plsc.parallel_loop(0, _N, 64, unroll=2,
                                    carry=(zero, zero, zero, zero))
                def accs(c, carry):
                    a0, a1, a2, a3 = carry
                    a0 = jnp.maximum(a0, jnp.abs(in_v[r, pl.ds(c, 16)]))
                    a1 = jnp.maximum(a1, jnp.abs(in_v[r, pl.ds(c + 16, 16)]))
                    a2 = jnp.maximum(a2, jnp.abs(in_v[r, pl.ds(c + 32, 16)]))
                    a3 = jnp.maximum(a3, jnp.abs(in_v[r, pl.ds(c + 48, 16)]))
                    return (a0, a1, a2, a3)

                acc = jnp.maximum(jnp.maximum(accs[0], accs[1]),
                                  jnp.maximum(accs[2], accs[3]))
                s = jnp.maximum(jnp.max(acc), 1e-6)
                sv = lax.broadcast(s, (16,))
                inv = 1.0 / sv

                @plsc.parallel_loop(0, _N, 16, unroll=8)
                def _q(c):
                    v = in_v[r, pl.ds(c, 16)]
                    xn = v * inv
                    cf = xn * jnp.float32(_HALF) + jnp.float32(_HALF)
                    cell = jnp.minimum(cf.astype(jnp.int32), _NCELL - 1)
                    t = plsc.load_gather(th_v, [cell])
                    lov = plsc.load_gather(lo_v, [cell])
                    hiv = plsc.load_gather(hi_v, [cell])
                    val = jnp.where(xn > t, hiv, lov)
                    out_v[r, pl.ds(c, 16)] = val * s

        pltpu.emit_pipeline(
            body,
            grid=(_N // _RCHUNK,),
            in_specs=[pl.BlockSpec((_RCHUNK, _N), lambda i: (i, 0))],
            out_specs=[pl.BlockSpec((_RCHUNK, _N), lambda i: (i, 0))],
            core_axis_name=("c", "s"),
            dimension_semantics=(pltpu.PARALLEL,),
        )(x_hbm, o_hbm)

    return sck(x, thresh, lo, hi)
